# baseline (speedup 1.0000x reference)
import jax
import jax.numpy as jnp
from jax import lax
from jax.experimental import pallas as pl
from jax.experimental.pallas import tpu as pltpu
from jax.experimental.pallas import tpu_sc as plsc

_L = 200
_B = 1024
_M = 64
_N = _L * _B
_SUB = 128
_UNIT = 1024
_NSUB = _UNIT // _SUB
_NW = 32
_ZROWS = _B // _NW


def _zero_fill(rows_v, outs, w):
    zvec = jnp.zeros((16,), jnp.float32)

    def _zrow(r, carry):
        for cc in range(_M // 16):
            rows_v[r, pl.ds(cc * 16, 16)] = zvec
        return carry

    lax.fori_loop(0, _ZROWS, _zrow, 0)
    zbase = w * _ZROWS
    for out_h in outs:
        pltpu.sync_copy(rows_v.at[pl.ds(0, _ZROWS)],
                        out_h.at[0, pl.ds(zbase, _ZROWS), :])


def _unit(idx_h, tab_h, out_h, idx_v, rows_v, sem, irow, l):
    pltpu.sync_copy(idx_h.at[pl.ds(irow, _NSUB)], idx_v)
    descs = [
        pltpu.async_copy(tab_h.at[idx_v.at[j]],
                         rows_v.at[pl.ds(j * _SUB, _SUB)], sem)
        for j in range(_NSUB)
    ]
    for d in descs:
        d.wait()
    pltpu.sync_copy(rows_v, out_h.at[l])


def _make_body(shifted):
    def _body(idx_h, tab_h, out_h, idx_v, rows_v, sem):
        w = lax.axis_index("s") * 2 + lax.axis_index("c")
        _zero_fill(rows_v, (out_h,), w)
        tot = _L - 1
        nu = (tot // _NW) + jnp.where(w < (tot % _NW), 1, 0)

        def _step(i, carry):
            l = 1 + w + i * _NW
            irow = (l - 1) * _NSUB if shifted else l * _NSUB
            _unit(idx_h, tab_h, out_h, idx_v, rows_v, sem, irow, l)
            return carry

        lax.fori_loop(0, nu, _step, 0)

    return _body


_sbody = _make_body(True)
_rbody = _make_body(False)


@jax.jit
def kernel(ly, lp, ry, emb_table, pos_table):
    ly2 = ly.astype(jnp.int32).reshape(_N // _SUB, _SUB)
    lp2 = lp.astype(jnp.int32).reshape(_N // _SUB, _SUB)
    ry2 = ry.astype(jnp.int32).reshape(_N // _SUB, _SUB)

    mesh = plsc.VectorSubcoreMesh(core_axis_name="c", subcore_axis_name="s")
    scratch = [
        pltpu.VMEM((_NSUB, _SUB), jnp.int32),
        pltpu.VMEM((_UNIT, _M), jnp.float32),
        pltpu.SemaphoreType.DMA,
    ]
    params = pltpu.CompilerParams(use_tc_tiling_on_sc=False)

    out1 = jax.ShapeDtypeStruct((_L, _B, _M), jnp.float32)

    def _call(body, name):
        return pl.kernel(
            body,
            mesh=mesh,
            out_type=out1,
            scratch_types=scratch,
            compiler_params=params,
            name=name,
        )

    po = _call(_sbody, "pos_gather")(lp2, pos_table)
    lo = _call(_sbody, "lemb_gather")(ly2, emb_table)
    ro = _call(_rbody, "remb_gather")(ry2, emb_table)
    return (lo, po, ro)
